# Initial kernel scaffold; baseline (speedup 1.0000x reference)
import jax
import jax.numpy as jnp
from jax.experimental import pallas as pl


def kernel(x, base_W, base_b, enc_W, W_dec, b_dec):
    raise NotImplementedError("write your pallas kernel here")


# speedup vs baseline: 5.1195x; 5.1195x over previous
import functools
import jax
import jax.numpy as jnp
from jax.experimental import pallas as pl
from jax.experimental.pallas import tpu as pltpu

K = 32


def _encode_body(x_ref, bw_ref, bb_ref, bd_ref, enc_ref, out_ref, r_s):
    @pl.when(pl.program_id(1) == 0)
    def _():
        r_s[...] = (
            jnp.dot(x_ref[...], bw_ref[...].T, preferred_element_type=jnp.float32)
            + bb_ref[...]
            - bd_ref[...]
        )

    out_ref[...] = jnp.dot(r_s[...], enc_ref[...].T, preferred_element_type=jnp.float32)


def _topk_body(pre_ref, ta_ref, ti_ref, *, lat):
    v = pre_ref[...]
    bt = v.shape[0]
    iota = jax.lax.broadcasted_iota(jnp.int32, (bt, lat), 1)
    kiota = jax.lax.broadcasted_iota(jnp.int32, (bt, K), 1)

    def step(k, carry):
        v, ta, ti = carry
        m = jnp.max(v, axis=1, keepdims=True)
        eq = v == m
        idx = jnp.min(jnp.where(eq, iota, lat), axis=1, keepdims=True)
        ta = jnp.where(kiota == k, m, ta)
        ti = jnp.where(kiota == k, idx, ti)
        v = jnp.where(iota == idx, -jnp.inf, v)
        return v, ta, ti

    ta0 = jnp.zeros((bt, K), jnp.float32)
    ti0 = jnp.zeros((bt, K), jnp.int32)
    _, ta, ti = jax.lax.fori_loop(0, K, step, (v, ta0, ti0))
    ta_ref[...] = ta
    ti_ref[...] = ti


def _decode_body(ta_ref, ti_ref, wd_ref, bd_ref, out_ref, acc_s, *, bl, n_l):
    l = pl.program_id(1)

    @pl.when(l == 0)
    def _():
        acc_s[...] = jnp.zeros_like(acc_s)

    ta = ta_ref[...]
    ti = ti_ref[...]
    bt = ta.shape[0]
    lane = jax.lax.broadcasted_iota(jnp.int32, (bt, bl), 1) + l * bl
    acts = jnp.zeros((bt, bl), jnp.float32)
    for k in range(K):
        acts += jnp.where(ti[:, k : k + 1] == lane, ta[:, k : k + 1], 0.0)
    acc_s[...] += jnp.dot(acts, wd_ref[...], preferred_element_type=jnp.float32)

    @pl.when(l == n_l - 1)
    def _():
        out_ref[...] = acc_s[...] + bd_ref[...]


def kernel(x, base_W, base_b, enc_W, W_dec, b_dec):
    b, s, d_in = x.shape
    t = b * s
    lat, d_out = enc_W.shape[0], W_dec.shape[1]
    x2 = x.reshape(t, d_in)
    bb2 = base_b.reshape(1, d_out)
    bd2 = b_dec.reshape(1, d_out)

    bt = min(256, t)
    bl = min(2048, lat)
    n_t, n_l = t // bt, lat // bl

    pre = pl.pallas_call(
        _encode_body,
        grid=(n_t, n_l),
        in_specs=[
            pl.BlockSpec((bt, d_in), lambda i, j: (i, 0)),
            pl.BlockSpec((d_out, d_in), lambda i, j: (0, 0)),
            pl.BlockSpec((1, d_out), lambda i, j: (0, 0)),
            pl.BlockSpec((1, d_out), lambda i, j: (0, 0)),
            pl.BlockSpec((bl, d_out), lambda i, j: (j, 0)),
        ],
        out_specs=pl.BlockSpec((bt, bl), lambda i, j: (i, j)),
        out_shape=jax.ShapeDtypeStruct((t, lat), jnp.float32),
        scratch_shapes=[pltpu.VMEM((bt, d_out), jnp.float32)],
    )(x2, base_W, bb2, bd2, enc_W)

    bt2 = min(64, t)
    ta, ti = pl.pallas_call(
        functools.partial(_topk_body, lat=lat),
        grid=(t // bt2,),
        in_specs=[pl.BlockSpec((bt2, lat), lambda i: (i, 0))],
        out_specs=[
            pl.BlockSpec((bt2, K), lambda i: (i, 0)),
            pl.BlockSpec((bt2, K), lambda i: (i, 0)),
        ],
        out_shape=[
            jax.ShapeDtypeStruct((t, K), jnp.float32),
            jax.ShapeDtypeStruct((t, K), jnp.int32),
        ],
    )(pre)

    out = pl.pallas_call(
        functools.partial(_decode_body, bl=bl, n_l=n_l),
        grid=(n_t, n_l),
        in_specs=[
            pl.BlockSpec((bt, K), lambda i, j: (i, 0)),
            pl.BlockSpec((bt, K), lambda i, j: (i, 0)),
            pl.BlockSpec((bl, d_out), lambda i, j: (j, 0)),
            pl.BlockSpec((1, d_out), lambda i, j: (0, 0)),
        ],
        out_specs=pl.BlockSpec((bt, d_out), lambda i, j: (i, 0)),
        out_shape=jax.ShapeDtypeStruct((t, d_out), jnp.float32),
        scratch_shapes=[pltpu.VMEM((bt, d_out), jnp.float32)],
    )(ta, ti, W_dec, bd2)

    return out.reshape(b, s, d_out)


# speedup vs baseline: 5.8324x; 1.1393x over previous
import functools
import jax
import jax.numpy as jnp
from jax import lax
from jax.experimental import pallas as pl
from jax.experimental.pallas import tpu as pltpu
from jax.experimental.pallas import tpu_sc as plsc

K = 32
_NC, _NS, _LANES = 2, 16, 16


def _encode_body(x_ref, bw_ref, bb_ref, bd_ref, enc_ref, out_ref, r_s):
    @pl.when(pl.program_id(1) == 0)
    def _():
        r_s[...] = (
            jnp.dot(x_ref[...], bw_ref[...].T, preferred_element_type=jnp.float32)
            + bb_ref[...]
            - bd_ref[...]
        )

    out_ref[...] = jnp.dot(r_s[...], enc_ref[...].T, preferred_element_type=jnp.float32)


def _topk_body(pre_ref, ta_ref, ti_ref, *, lat):
    v = pre_ref[...]
    bt = v.shape[0]
    iota = jax.lax.broadcasted_iota(jnp.int32, (bt, lat), 1)
    kiota = jax.lax.broadcasted_iota(jnp.int32, (bt, K), 1)

    def step(k, carry):
        v, ta, ti = carry
        m = jnp.max(v, axis=1, keepdims=True)
        eq = v == m
        idx = jnp.min(jnp.where(eq, iota, lat), axis=1, keepdims=True)
        ta = jnp.where(kiota == k, m, ta)
        ti = jnp.where(kiota == k, idx, ti)
        v = jnp.where(iota == idx, -jnp.inf, v)
        return v, ta, ti

    ta0 = jnp.zeros((bt, K), jnp.float32)
    ti0 = jnp.zeros((bt, K), jnp.int32)
    _, ta, ti = jax.lax.fori_loop(0, K, step, (v, ta0, ti0))
    ta_ref[...] = ta
    ti_ref[...] = ti


def _sc_decode_body(ta_hbm, ti_hbm, wd_hbm, bd_hbm, out_hbm,
                    ti_v, ta_v, rows_v, bd_v, out_v, sem):
    nw = _NC * _NS
    wid = lax.axis_index("s") * _NC + lax.axis_index("c")
    t = ta_hbm.shape[0]
    tpw = t // nw
    base = wid * tpw
    d = wd_hbm.shape[1]
    nch = d // _LANES

    pltpu.sync_copy(ti_hbm.at[pl.ds(base, tpw)], ti_v)
    pltpu.sync_copy(ta_hbm.at[pl.ds(base, tpw)], ta_v)
    pltpu.sync_copy(bd_hbm, bd_v)

    def tok_body(j, _):
        pltpu.async_copy(wd_hbm.at[ti_v.at[j]], rows_v, sem).wait()
        tvecs = [ta_v[j, pl.ds(i * _LANES, _LANES)] for i in range(K // _LANES)]
        tak = [tvecs[k // _LANES][k % _LANES] for k in range(K)]

        def ch_body(c, _):
            sl = pl.ds(c * _LANES, _LANES)
            acc = bd_v[sl]
            for k in range(K):
                acc = acc + tak[k] * rows_v[k, sl]
            out_v[j, sl] = acc
            return 0

        lax.fori_loop(0, nch, ch_body, 0)
        return 0

    lax.fori_loop(0, tpw, tok_body, 0)
    pltpu.sync_copy(out_v, out_hbm.at[pl.ds(base, tpw)])


def _decode_body(ta_ref, ti_ref, wd_ref, bd_ref, out_ref, acc_s, *, bl, n_l):
    l = pl.program_id(1)

    @pl.when(l == 0)
    def _():
        acc_s[...] = jnp.zeros_like(acc_s)

    ta = ta_ref[...]
    ti = ti_ref[...]
    bt = ta.shape[0]
    lane = jax.lax.broadcasted_iota(jnp.int32, (bt, bl), 1) + l * bl
    acts = jnp.zeros((bt, bl), jnp.float32)
    for k in range(K):
        acts += jnp.where(ti[:, k : k + 1] == lane, ta[:, k : k + 1], 0.0)
    acc_s[...] += jnp.dot(acts, wd_ref[...], preferred_element_type=jnp.float32)

    @pl.when(l == n_l - 1)
    def _():
        out_ref[...] = acc_s[...] + bd_ref[...]


def kernel(x, base_W, base_b, enc_W, W_dec, b_dec):
    b, s, d_in = x.shape
    t = b * s
    lat, d_out = enc_W.shape[0], W_dec.shape[1]
    x2 = x.reshape(t, d_in)
    bb2 = base_b.reshape(1, d_out)
    bd2 = b_dec.reshape(1, d_out)

    bt = min(256, t)
    bl = min(2048, lat)
    n_t, n_l = t // bt, lat // bl

    pre = pl.pallas_call(
        _encode_body,
        grid=(n_t, n_l),
        in_specs=[
            pl.BlockSpec((bt, d_in), lambda i, j: (i, 0)),
            pl.BlockSpec((d_out, d_in), lambda i, j: (0, 0)),
            pl.BlockSpec((1, d_out), lambda i, j: (0, 0)),
            pl.BlockSpec((1, d_out), lambda i, j: (0, 0)),
            pl.BlockSpec((bl, d_out), lambda i, j: (j, 0)),
        ],
        out_specs=pl.BlockSpec((bt, bl), lambda i, j: (i, j)),
        out_shape=jax.ShapeDtypeStruct((t, lat), jnp.float32),
        scratch_shapes=[pltpu.VMEM((bt, d_out), jnp.float32)],
    )(x2, base_W, bb2, bd2, enc_W)

    bt2 = min(64, t)
    ta, ti = pl.pallas_call(
        functools.partial(_topk_body, lat=lat),
        grid=(t // bt2,),
        in_specs=[pl.BlockSpec((bt2, lat), lambda i: (i, 0))],
        out_specs=[
            pl.BlockSpec((bt2, K), lambda i: (i, 0)),
            pl.BlockSpec((bt2, K), lambda i: (i, 0)),
        ],
        out_shape=[
            jax.ShapeDtypeStruct((t, K), jnp.float32),
            jax.ShapeDtypeStruct((t, K), jnp.int32),
        ],
    )(pre)

    tpw = t // (_NC * _NS)
    mesh = plsc.VectorSubcoreMesh(core_axis_name="c", subcore_axis_name="s")
    out = pl.kernel(
        _sc_decode_body,
        out_type=jax.ShapeDtypeStruct((t, d_out), jnp.float32),
        mesh=mesh,
        scratch_types=[
            pltpu.VMEM((tpw, K), jnp.int32),
            pltpu.VMEM((tpw, K), jnp.float32),
            pltpu.VMEM((K, d_out), jnp.float32),
            pltpu.VMEM((d_out,), jnp.float32),
            pltpu.VMEM((tpw, d_out), jnp.float32),
            pltpu.SemaphoreType.DMA,
        ],
    )(ta, ti, W_dec, b_dec)

    return out.reshape(b, s, d_out)


# speedup vs baseline: 25.8752x; 4.4365x over previous
import functools
import jax
import jax.numpy as jnp
from jax import lax
from jax.experimental import pallas as pl
from jax.experimental.pallas import tpu as pltpu
from jax.experimental.pallas import tpu_sc as plsc

K = 32
_NC, _NS, _LANES = 2, 16, 16


def _encode_body(x_ref, bw_ref, bb_ref, bd_ref, enc_ref, out_ref, cm_ref, r_s):
    @pl.when(pl.program_id(1) == 0)
    def _():
        r_s[...] = (
            jnp.dot(x_ref[...], bw_ref[...].T, preferred_element_type=jnp.float32)
            + bb_ref[...]
            - bd_ref[...]
        )

    pre = jnp.dot(r_s[...], enc_ref[...].T, preferred_element_type=jnp.float32)
    out_ref[...] = pre
    n_sub = pre.shape[1] // 128
    for c in range(n_sub):
        cm_ref[0, :, c : c + 1] = jnp.max(
            pre[:, c * 128 : (c + 1) * 128], axis=1, keepdims=True
        )


def _topk_body(pre_ref, ta_ref, ti_ref, *, lat):
    v = pre_ref[...]
    bt = v.shape[0]
    iota = jax.lax.broadcasted_iota(jnp.int32, (bt, lat), 1)
    kiota = jax.lax.broadcasted_iota(jnp.int32, (bt, K), 1)

    def step(k, carry):
        v, ta, ti = carry
        m = jnp.max(v, axis=1, keepdims=True)
        eq = v == m
        idx = jnp.min(jnp.where(eq, iota, lat), axis=1, keepdims=True)
        ta = jnp.where(kiota == k, m, ta)
        ti = jnp.where(kiota == k, idx, ti)
        v = jnp.where(iota == idx, -jnp.inf, v)
        return v, ta, ti

    ta0 = jnp.zeros((bt, K), jnp.float32)
    ti0 = jnp.zeros((bt, K), jnp.int32)
    _, ta, ti = jax.lax.fori_loop(0, K, step, (v, ta0, ti0))
    ta_ref[...] = ta
    ti_ref[...] = ti


def _argmax_butterfly(val, idx, vt_ref, it_ref):
    for sh in (8, 4, 2, 1):
        vt_ref[pl.ds(0, _LANES)] = val
        it_ref[pl.ds(0, _LANES)] = idx
        vs = vt_ref[pl.ds(sh, _LANES)]
        is_ = it_ref[pl.ds(sh, _LANES)]
        better = (vs > val) | ((vs == val) & (is_ < idx))
        val = jnp.where(better, vs, val)
        idx = jnp.where(better, is_, idx)
    return val[0], idx[0]


def _max_butterfly(val, vt_ref):
    for sh in (8, 4, 2, 1):
        vt_ref[pl.ds(0, _LANES)] = val
        val = jnp.maximum(val, vt_ref[pl.ds(sh, _LANES)])
    return val[0]


def _sc_topk_decode_body(pre_hbm, cm_hbm, wd_hbm, bd_hbm, out_hbm,
                         row_v, cmst_v, cmw_v, idx_v, vals_v, rows_v, bd_v,
                         out_v, vt_v, it_v, rsem_a, rsem_b, gsem):
    nw = _NC * _NS
    wid = lax.axis_index("s") * _NC + lax.axis_index("c")
    t = pre_hbm.shape[0]
    tpw = t // nw
    base = wid * tpw
    lat = pre_hbm.shape[1]
    d = wd_hbm.shape[1]
    ncm = lat // 128
    nv2 = ncm // _LANES

    lane = lax.broadcasted_iota(jnp.int32, (_LANES,), 0)
    neg_inf = jnp.float32(-jnp.inf)
    rsems = [rsem_a, rsem_b]

    vt_v[pl.ds(_LANES, _LANES)] = jnp.full((_LANES,), neg_inf, jnp.float32)
    it_v[pl.ds(_LANES, _LANES)] = jnp.full((_LANES,), 2147483647, jnp.int32)

    pltpu.sync_copy(bd_hbm, bd_v)
    for b in range(2):
        pltpu.async_copy(pre_hbm.at[pl.ds(base + b, 1)], row_v.at[pl.ds(b, 1)],
                         rsems[b])
        pltpu.async_copy(cm_hbm.at[:, pl.ds(base + b, 1), :],
                         cmst_v.at[:, pl.ds(b, 1), :], rsems[b])

    def tok_pair(jp, _):
        for b in range(2):
            j = jp * 2 + b
            tok = base + j
            pltpu.make_async_copy(pre_hbm.at[pl.ds(tok, 1)],
                                  row_v.at[pl.ds(b, 1)], rsems[b]).wait()
            pltpu.make_async_copy(cm_hbm.at[:, pl.ds(tok, 1), :],
                                  cmst_v.at[:, pl.ds(b, 1), :], rsems[b]).wait()
            for r in range(nv2):
                cmw_v[b, pl.ds(r * _LANES, _LANES)] = cmst_v[r, b, pl.ds(0, _LANES)]

            def extract(kk, _):
                val = cmw_v[b, pl.ds(0, _LANES)]
                idx = lane
                for r in range(1, nv2):
                    c = cmw_v[b, pl.ds(r * _LANES, _LANES)]
                    better = c > val
                    val = jnp.where(better, c, val)
                    idx = jnp.where(better, lane + r * _LANES, idx)
                gmax, cstar = _argmax_butterfly(val, idx, vt_v, it_v)
                cbase = cstar * 128
                v0 = row_v[b, pl.ds(cbase, _LANES)]
                iv0 = cbase + lane
                for i in range(1, 8):
                    c = row_v[b, pl.ds(cbase + i * _LANES, _LANES)]
                    better = c > v0
                    v0 = jnp.where(better, c, v0)
                    iv0 = jnp.where(better, cbase + i * _LANES + lane, iv0)
                _gv, gidx = _argmax_butterfly(v0, iv0, vt_v, it_v)
                roff = (kk // _LANES) * _LANES
                rpos = kk - roff
                idx_v[pl.ds(roff, _LANES)] = jnp.where(
                    lane == rpos, gidx, idx_v[pl.ds(roff, _LANES)])
                vals_v[pl.ds(roff, _LANES)] = jnp.where(
                    lane == rpos, gmax, vals_v[pl.ds(roff, _LANES)])
                off = (gidx // _LANES) * _LANES
                pos = gidx - off
                row_v[b, pl.ds(off, _LANES)] = jnp.where(
                    lane == pos, neg_inf, row_v[b, pl.ds(off, _LANES)])
                nm = row_v[b, pl.ds(cbase, _LANES)]
                for i in range(1, 8):
                    nm = jnp.maximum(nm, row_v[b, pl.ds(cbase + i * _LANES, _LANES)])
                newmax = _max_butterfly(nm, vt_v)
                coff = (cstar // _LANES) * _LANES
                cpos = cstar - coff
                cmw_v[b, pl.ds(coff, _LANES)] = jnp.where(
                    lane == cpos, newmax, cmw_v[b, pl.ds(coff, _LANES)])
                return 0

            lax.fori_loop(0, K, extract, 0)

            gcp = pltpu.async_copy(wd_hbm.at[idx_v], rows_v, gsem)
            jn = jnp.minimum(tok + 2, base + tpw - 1)
            pltpu.async_copy(pre_hbm.at[pl.ds(jn, 1)], row_v.at[pl.ds(b, 1)],
                             rsems[b])
            pltpu.async_copy(cm_hbm.at[:, pl.ds(jn, 1), :],
                             cmst_v.at[:, pl.ds(b, 1), :], rsems[b])
            gcp.wait()

            tv = [vals_v[pl.ds(i * _LANES, _LANES)] for i in range(K // _LANES)]
            tak = [tv[k // _LANES][k % _LANES] for k in range(K)]

            def ch_body(c, _):
                sl = pl.ds(c * _LANES, _LANES)
                acc = bd_v[sl]
                for k in range(K):
                    acc = acc + tak[k] * rows_v[k, sl]
                out_v[j, sl] = acc
                return 0

            lax.fori_loop(0, d // _LANES, ch_body, 0)
        return 0

    lax.fori_loop(0, tpw // 2, tok_pair, 0)
    for b in range(2):
        pltpu.make_async_copy(pre_hbm.at[pl.ds(base, 1)],
                              row_v.at[pl.ds(b, 1)], rsems[b]).wait()
        pltpu.make_async_copy(cm_hbm.at[:, pl.ds(base, 1), :],
                              cmst_v.at[:, pl.ds(b, 1), :], rsems[b]).wait()
    pltpu.sync_copy(out_v, out_hbm.at[pl.ds(base, tpw)])


def _decode_body(ta_ref, ti_ref, wd_ref, bd_ref, out_ref, acc_s, *, bl, n_l):
    l = pl.program_id(1)

    @pl.when(l == 0)
    def _():
        acc_s[...] = jnp.zeros_like(acc_s)

    ta = ta_ref[...]
    ti = ti_ref[...]
    bt = ta.shape[0]
    lane = jax.lax.broadcasted_iota(jnp.int32, (bt, bl), 1) + l * bl
    acts = jnp.zeros((bt, bl), jnp.float32)
    for k in range(K):
        acts += jnp.where(ti[:, k : k + 1] == lane, ta[:, k : k + 1], 0.0)
    acc_s[...] += jnp.dot(acts, wd_ref[...], preferred_element_type=jnp.float32)

    @pl.when(l == n_l - 1)
    def _():
        out_ref[...] = acc_s[...] + bd_ref[...]


def kernel(x, base_W, base_b, enc_W, W_dec, b_dec):
    b, s, d_in = x.shape
    t = b * s
    lat, d_out = enc_W.shape[0], W_dec.shape[1]
    x2 = x.reshape(t, d_in)
    bb2 = base_b.reshape(1, d_out)
    bd2 = b_dec.reshape(1, d_out)

    bt = min(256, t)
    bl = min(2048, lat)
    n_t, n_l = t // bt, lat // bl

    pre, cm = pl.pallas_call(
        _encode_body,
        grid=(n_t, n_l),
        in_specs=[
            pl.BlockSpec((bt, d_in), lambda i, j: (i, 0)),
            pl.BlockSpec((d_out, d_in), lambda i, j: (0, 0)),
            pl.BlockSpec((1, d_out), lambda i, j: (0, 0)),
            pl.BlockSpec((1, d_out), lambda i, j: (0, 0)),
            pl.BlockSpec((bl, d_out), lambda i, j: (j, 0)),
        ],
        out_specs=[
            pl.BlockSpec((bt, bl), lambda i, j: (i, j)),
            pl.BlockSpec((1, bt, bl // 128), lambda i, j: (j, i, 0)),
        ],
        out_shape=[
            jax.ShapeDtypeStruct((t, lat), jnp.float32),
            jax.ShapeDtypeStruct((n_l, t, bl // 128), jnp.float32),
        ],
        scratch_shapes=[pltpu.VMEM((bt, d_out), jnp.float32)],
    )(x2, base_W, bb2, bd2, enc_W)

    tpw = t // (_NC * _NS)
    mesh = plsc.VectorSubcoreMesh(core_axis_name="c", subcore_axis_name="s")
    out = pl.kernel(
        _sc_topk_decode_body,
        out_type=jax.ShapeDtypeStruct((t, d_out), jnp.float32),
        mesh=mesh,
        scratch_types=[
            pltpu.VMEM((2, lat), jnp.float32),
            pltpu.VMEM((n_l, 2, bl // 128), jnp.float32),
            pltpu.VMEM((2, lat // 128), jnp.float32),
            pltpu.VMEM((K,), jnp.int32),
            pltpu.VMEM((K,), jnp.float32),
            pltpu.VMEM((K, d_out), jnp.float32),
            pltpu.VMEM((d_out,), jnp.float32),
            pltpu.VMEM((tpw, d_out), jnp.float32),
            pltpu.VMEM((2 * _LANES,), jnp.float32),
            pltpu.VMEM((2 * _LANES,), jnp.int32),
            pltpu.SemaphoreType.DMA,
            pltpu.SemaphoreType.DMA,
            pltpu.SemaphoreType.DMA,
        ],
    )(pre, cm, W_dec, b_dec)

    return out.reshape(b, s, d_out)


# speedup vs baseline: 26.3533x; 1.0185x over previous
import functools
import jax
import jax.numpy as jnp
from jax import lax
from jax.experimental import pallas as pl
from jax.experimental.pallas import tpu as pltpu
from jax.experimental.pallas import tpu_sc as plsc

K = 32
_NC, _NS, _LANES = 2, 16, 16


def _encode_body(x_ref, bw_ref, bb_ref, bd_ref, enc_ref, out_ref, cm_ref, r_s):
    @pl.when(pl.program_id(1) == 0)
    def _():
        r_s[...] = (
            jnp.dot(x_ref[...], bw_ref[...].T, preferred_element_type=jnp.float32)
            + bb_ref[...]
            - bd_ref[...]
        )

    pre = jnp.dot(r_s[...], enc_ref[...].T, preferred_element_type=jnp.float32)
    out_ref[...] = pre
    n_sub = pre.shape[1] // 128
    for c in range(n_sub):
        cm_ref[0, :, c : c + 1] = jnp.max(
            pre[:, c * 128 : (c + 1) * 128], axis=1, keepdims=True
        )


def _topk_body(pre_ref, ta_ref, ti_ref, *, lat):
    v = pre_ref[...]
    bt = v.shape[0]
    iota = jax.lax.broadcasted_iota(jnp.int32, (bt, lat), 1)
    kiota = jax.lax.broadcasted_iota(jnp.int32, (bt, K), 1)

    def step(k, carry):
        v, ta, ti = carry
        m = jnp.max(v, axis=1, keepdims=True)
        eq = v == m
        idx = jnp.min(jnp.where(eq, iota, lat), axis=1, keepdims=True)
        ta = jnp.where(kiota == k, m, ta)
        ti = jnp.where(kiota == k, idx, ti)
        v = jnp.where(iota == idx, -jnp.inf, v)
        return v, ta, ti

    ta0 = jnp.zeros((bt, K), jnp.float32)
    ti0 = jnp.zeros((bt, K), jnp.int32)
    _, ta, ti = jax.lax.fori_loop(0, K, step, (v, ta0, ti0))
    ta_ref[...] = ta
    ti_ref[...] = ti


def _argmax_butterfly(val, idx, vt_ref, it_ref):
    for sh in (8, 4, 2, 1):
        vt_ref[pl.ds(0, _LANES)] = val
        it_ref[pl.ds(0, _LANES)] = idx
        vs = vt_ref[pl.ds(sh, _LANES)]
        is_ = it_ref[pl.ds(sh, _LANES)]
        better = (vs > val) | ((vs == val) & (is_ < idx))
        val = jnp.where(better, vs, val)
        idx = jnp.where(better, is_, idx)
    return val[0], idx[0]


def _max_butterfly(val, vt_ref):
    for sh in (8, 4, 2, 1):
        vt_ref[pl.ds(0, _LANES)] = val
        val = jnp.maximum(val, vt_ref[pl.ds(sh, _LANES)])
    return val[0]


def _argmax_tree(pairs):
    while len(pairs) > 1:
        nxt = []
        for (va, ia), (vb, ib) in zip(pairs[0::2], pairs[1::2]):
            better = (vb > va) | ((vb == va) & (ib < ia))
            nxt.append((jnp.where(better, vb, va), jnp.where(better, ib, ia)))
        if len(pairs) % 2:
            nxt.append(pairs[-1])
        pairs = nxt
    return pairs[0]


def _sc_topk_decode_body(pre_hbm, cm_hbm, wd_hbm, bd_hbm, out_hbm,
                         row_v, cmst_v, cmw_v, idx_v, vals_v, rows_v, bd_v,
                         orow_v, vt_v, it_v, rsem_a, rsem_b, gsem_a, gsem_b):
    nw = _NC * _NS
    wid = lax.axis_index("s") * _NC + lax.axis_index("c")
    t = pre_hbm.shape[0]
    tpw = t // nw
    base = wid * tpw
    lat = pre_hbm.shape[1]
    d = wd_hbm.shape[1]
    ncm = lat // 128
    nv2 = ncm // _LANES

    lane = lax.broadcasted_iota(jnp.int32, (_LANES,), 0)
    neg_inf = jnp.float32(-jnp.inf)
    rsems = [rsem_a, rsem_b]
    gsems = [gsem_a, gsem_b]

    for b in range(2):
        vt_v[b, pl.ds(_LANES, _LANES)] = jnp.full((_LANES,), neg_inf, jnp.float32)
        it_v[b, pl.ds(_LANES, _LANES)] = jnp.full((_LANES,), 2147483647, jnp.int32)

    pltpu.sync_copy(bd_hbm, bd_v)
    for b in range(2):
        pltpu.async_copy(pre_hbm.at[pl.ds(base + b, 1)], row_v.at[pl.ds(b, 1)],
                         rsems[b])
        pltpu.async_copy(cm_hbm.at[:, pl.ds(base + b, 1), :],
                         cmst_v.at[:, pl.ds(b, 1), :], rsems[b])

    def tok_pair(jp, _):
        j0 = jp * 2
        for b in range(2):
            tok = base + j0 + b
            pltpu.make_async_copy(pre_hbm.at[pl.ds(tok, 1)],
                                  row_v.at[pl.ds(b, 1)], rsems[b]).wait()
            pltpu.make_async_copy(cm_hbm.at[:, pl.ds(tok, 1), :],
                                  cmst_v.at[:, pl.ds(b, 1), :], rsems[b]).wait()
            for r in range(nv2):
                cmw_v[b, pl.ds(r * _LANES, _LANES)] = cmst_v[r, b, pl.ds(0, _LANES)]

        def extract(kk, _):
            for b in range(2):
                pairs = [(cmw_v[b, pl.ds(r * _LANES, _LANES)], lane + r * _LANES)
                         for r in range(nv2)]
                val, idx = _argmax_tree(pairs)
                gmax, cstar = _argmax_butterfly(val, idx, vt_v.at[b], it_v.at[b])
                cbase = cstar * 128
                pairs = [(row_v[b, pl.ds(cbase + i * _LANES, _LANES)],
                          cbase + i * _LANES + lane) for i in range(8)]
                v0, iv0 = _argmax_tree(pairs)
                _gv, gidx = _argmax_butterfly(v0, iv0, vt_v.at[b], it_v.at[b])
                roff = (kk // _LANES) * _LANES
                rpos = kk - roff
                idx_v[b, pl.ds(roff, _LANES)] = jnp.where(
                    lane == rpos, gidx, idx_v[b, pl.ds(roff, _LANES)])
                vals_v[b, pl.ds(roff, _LANES)] = jnp.where(
                    lane == rpos, gmax, vals_v[b, pl.ds(roff, _LANES)])
                off = (gidx // _LANES) * _LANES
                pos = gidx - off
                row_v[b, pl.ds(off, _LANES)] = jnp.where(
                    lane == pos, neg_inf, row_v[b, pl.ds(off, _LANES)])
                nm = row_v[b, pl.ds(cbase, _LANES)]
                for i in range(1, 8):
                    nm = jnp.maximum(nm, row_v[b, pl.ds(cbase + i * _LANES, _LANES)])
                newmax = _max_butterfly(nm, vt_v.at[b])
                coff = (cstar // _LANES) * _LANES
                cpos = cstar - coff
                cmw_v[b, pl.ds(coff, _LANES)] = jnp.where(
                    lane == cpos, newmax, cmw_v[b, pl.ds(coff, _LANES)])
            return 0

        lax.fori_loop(0, K, extract, 0)

        for b in range(2):
            pltpu.async_copy(wd_hbm.at[idx_v.at[b]], rows_v.at[b], gsems[b])
        for b in range(2):
            jn = jnp.minimum(base + j0 + b + 2, base + tpw - 1)
            pltpu.async_copy(pre_hbm.at[pl.ds(jn, 1)], row_v.at[pl.ds(b, 1)],
                             rsems[b])
            pltpu.async_copy(cm_hbm.at[:, pl.ds(jn, 1), :],
                             cmst_v.at[:, pl.ds(b, 1), :], rsems[b])

        for b in range(2):
            tok = base + j0 + b
            pltpu.make_async_copy(wd_hbm.at[idx_v.at[b]],
                                  rows_v.at[b], gsems[b]).wait()
            tv = [vals_v[b, pl.ds(i * _LANES, _LANES)] for i in range(K // _LANES)]
            tak = [tv[k // _LANES][k % _LANES] for k in range(K)]

            def ch_body(c, _):
                sl = pl.ds(c * _LANES, _LANES)
                acc = bd_v[sl]
                for k in range(K):
                    acc = acc + tak[k] * rows_v[b, k, sl]
                orow_v[b, sl] = acc
                return 0

            lax.fori_loop(0, d // _LANES, ch_body, 0)
            pltpu.sync_copy(orow_v.at[pl.ds(b, 1)], out_hbm.at[pl.ds(tok, 1)])
        return 0

    lax.fori_loop(0, tpw // 2, tok_pair, 0)
    for b in range(2):
        pltpu.make_async_copy(pre_hbm.at[pl.ds(base, 1)],
                              row_v.at[pl.ds(b, 1)], rsems[b]).wait()
        pltpu.make_async_copy(cm_hbm.at[:, pl.ds(base, 1), :],
                              cmst_v.at[:, pl.ds(b, 1), :], rsems[b]).wait()


def _decode_body(ta_ref, ti_ref, wd_ref, bd_ref, out_ref, acc_s, *, bl, n_l):
    l = pl.program_id(1)

    @pl.when(l == 0)
    def _():
        acc_s[...] = jnp.zeros_like(acc_s)

    ta = ta_ref[...]
    ti = ti_ref[...]
    bt = ta.shape[0]
    lane = jax.lax.broadcasted_iota(jnp.int32, (bt, bl), 1) + l * bl
    acts = jnp.zeros((bt, bl), jnp.float32)
    for k in range(K):
        acts += jnp.where(ti[:, k : k + 1] == lane, ta[:, k : k + 1], 0.0)
    acc_s[...] += jnp.dot(acts, wd_ref[...], preferred_element_type=jnp.float32)

    @pl.when(l == n_l - 1)
    def _():
        out_ref[...] = acc_s[...] + bd_ref[...]


def kernel(x, base_W, base_b, enc_W, W_dec, b_dec):
    b, s, d_in = x.shape
    t = b * s
    lat, d_out = enc_W.shape[0], W_dec.shape[1]
    x2 = x.reshape(t, d_in)
    bb2 = base_b.reshape(1, d_out)
    bd2 = b_dec.reshape(1, d_out)

    bt = min(256, t)
    bl = min(2048, lat)
    n_t, n_l = t // bt, lat // bl

    pre, cm = pl.pallas_call(
        _encode_body,
        grid=(n_t, n_l),
        in_specs=[
            pl.BlockSpec((bt, d_in), lambda i, j: (i, 0)),
            pl.BlockSpec((d_out, d_in), lambda i, j: (0, 0)),
            pl.BlockSpec((1, d_out), lambda i, j: (0, 0)),
            pl.BlockSpec((1, d_out), lambda i, j: (0, 0)),
            pl.BlockSpec((bl, d_out), lambda i, j: (j, 0)),
        ],
        out_specs=[
            pl.BlockSpec((bt, bl), lambda i, j: (i, j)),
            pl.BlockSpec((1, bt, bl // 128), lambda i, j: (j, i, 0)),
        ],
        out_shape=[
            jax.ShapeDtypeStruct((t, lat), jnp.float32),
            jax.ShapeDtypeStruct((n_l, t, bl // 128), jnp.float32),
        ],
        scratch_shapes=[pltpu.VMEM((bt, d_out), jnp.float32)],
    )(x2, base_W, bb2, bd2, enc_W)

    tpw = t // (_NC * _NS)
    mesh = plsc.VectorSubcoreMesh(core_axis_name="c", subcore_axis_name="s")
    out = pl.kernel(
        _sc_topk_decode_body,
        out_type=jax.ShapeDtypeStruct((t, d_out), jnp.float32),
        mesh=mesh,
        scratch_types=[
            pltpu.VMEM((2, lat), jnp.float32),
            pltpu.VMEM((n_l, 2, bl // 128), jnp.float32),
            pltpu.VMEM((2, lat // 128), jnp.float32),
            pltpu.VMEM((2, K), jnp.int32),
            pltpu.VMEM((2, K), jnp.float32),
            pltpu.VMEM((2, K, d_out), jnp.float32),
            pltpu.VMEM((d_out,), jnp.float32),
            pltpu.VMEM((2, d_out), jnp.float32),
            pltpu.VMEM((2, 2 * _LANES), jnp.float32),
            pltpu.VMEM((2, 2 * _LANES), jnp.int32),
            pltpu.SemaphoreType.DMA,
            pltpu.SemaphoreType.DMA,
            pltpu.SemaphoreType.DMA,
            pltpu.SemaphoreType.DMA,
        ],
    )(pre, cm, W_dec, b_dec)

    return out.reshape(b, s, d_out)
